# baseline (speedup 1.0000x reference)
import functools

import jax
import jax.numpy as jnp
import numpy as np
from jax.experimental import pallas as pl
from jax.experimental.pallas import tpu as pltpu

_NB_SAMPLES = 16
_GAMMA = np.float32(0.04)
_EPS = np.float32(1e-10)
_LO = np.float32(np.nextafter(np.float32(-1.0), np.float32(0.0)))
_SPAN = np.float32(np.float32(1.0) - _LO)
_SQRT2 = np.float32(np.sqrt(2.0))

_KS0 = np.uint32(0)
_KS1 = np.uint32(1)
_KS2 = np.uint32(0x1BD11BDA) ^ _KS0 ^ _KS1
_ROT = (13, 15, 26, 6, 17, 29, 16, 24)


def _threefry_bits(x1):
    ks = (_KS0, _KS1, _KS2)
    x0 = jnp.zeros_like(x1) + _KS0
    x1 = x1 + _KS1
    for i in range(5):
        rs = _ROT[:4] if i % 2 == 0 else _ROT[4:]
        for r in rs:
            x0 = x0 + x1
            x1 = (x1 << jnp.uint32(r)) | (x1 >> jnp.uint32(32 - r))
            x1 = x0 ^ x1
        x0 = x0 + ks[(i + 1) % 3]
        x1 = x1 + ks[(i + 2) % 3] + jnp.uint32(i + 1)
    return x0 ^ x1


def _noise(idx_u32):
    bits = _threefry_bits(idx_u32)
    float_bits = (bits >> jnp.uint32(9)) | jnp.uint32(0x3F800000)
    f = jax.lax.bitcast_convert_type(float_bits, jnp.float32) - jnp.float32(1.0)
    u = jnp.maximum(_LO, f * _SPAN + _LO)
    return _SQRT2 * jax.lax.erf_inv(u)


def _body(zfar_ref, znear_ref, zb_ref, pm_ref, mk_ref, out_ref, *, rt, sstride):
    k = zb_ref.shape[0]
    s = pl.program_id(1)

    zfar = zfar_ref[0]
    denom = zfar - znear_ref[0]

    z_inv = [(zfar - zb_ref[c]) / denom * mk_ref[c] for c in range(k)]
    zmax = z_inv[0]
    for c in range(1, k):
        zmax = jnp.maximum(zmax, z_inv[c])
    zmax = jnp.maximum(zmax, _EPS)
    z_map = [_GAMMA * jnp.log(pm_ref[c]) + z_inv[c] - zmax for c in range(k)]
    z_map.append(_EPS - zmax)

    row = jax.lax.broadcasted_iota(jnp.int32, (rt, 128), 0)
    lane = jax.lax.broadcasted_iota(jnp.int32, (rt, 128), 1)
    p = (pl.program_id(0) * rt + row) * 128 + lane
    base = (p * (k + 1) + s * sstride).astype(jnp.uint32)

    best = None
    bidx = None
    for c in range(k + 1):
        n = _noise(base + jnp.uint32(c))
        v = z_map[c] + _GAMMA * n
        if c == 0:
            best = v
            bidx = jnp.zeros_like(row)
        else:
            upd = v > best
            best = jnp.where(upd, v, best)
            bidx = jnp.where(upd, c, bidx)

    last = jnp.int32(_NB_SAMPLES - 1)
    for c in range(k + 1):
        w = (bidx == c).astype(jnp.float32)

        @pl.when(s == 0)
        def _():
            out_ref[c] = w

        @pl.when(jnp.logical_and(s > 0, s < last))
        def _():
            out_ref[c] += w

        @pl.when(s == last)
        def _():
            out_ref[c] = (out_ref[c] + w) * jnp.float32(1.0 / _NB_SAMPLES)


def kernel(zbuf, zfar, znear, prob_map, mask):
    b, h, w, k = zbuf.shape
    p = b * h * w
    r = p // 128
    rt = 56 if r % 56 == 0 else 8
    zb = zbuf.reshape(p, k).T.reshape(k, r, 128)
    pm = prob_map.reshape(p, k).T.reshape(k, r, 128)
    mk = mask.reshape(p, k).T.reshape(k, r, 128)

    body = functools.partial(_body, rt=rt, sstride=p * (k + 1))
    out = pl.pallas_call(
        body,
        grid=(r // rt, _NB_SAMPLES),
        in_specs=[
            pl.BlockSpec(memory_space=pltpu.SMEM),
            pl.BlockSpec(memory_space=pltpu.SMEM),
            pl.BlockSpec((k, rt, 128), lambda i, s: (0, i, 0)),
            pl.BlockSpec((k, rt, 128), lambda i, s: (0, i, 0)),
            pl.BlockSpec((k, rt, 128), lambda i, s: (0, i, 0)),
        ],
        out_specs=pl.BlockSpec((k + 1, rt, 128), lambda i, s: (0, i, 0)),
        out_shape=jax.ShapeDtypeStruct((k + 1, r, 128), jnp.float32),
    )(zfar, znear, zb, pm, mk)
    return out.reshape(k + 1, p).T.reshape(b, h, w, k + 1)


# speedup vs baseline: 1.0338x; 1.0338x over previous
import functools

import jax
import jax.numpy as jnp
import numpy as np
from jax.experimental import pallas as pl
from jax.experimental.pallas import tpu as pltpu

_NB_SAMPLES = 16
_GAMMA = np.float32(0.04)
_EPS = np.float32(1e-10)
_LO = np.float32(np.nextafter(np.float32(-1.0), np.float32(0.0)))
_SPAN = np.float32(np.float32(1.0) - _LO)
_SQRT2 = np.float32(np.sqrt(2.0))

_KS0 = np.uint32(0)
_KS1 = np.uint32(1)
_KS2 = np.uint32(0x1BD11BDA) ^ _KS0 ^ _KS1
_ROT = (13, 15, 26, 6, 17, 29, 16, 24)


def _threefry_bits(x1):
    ks = (_KS0, _KS1, _KS2)
    x0 = jnp.zeros_like(x1) + _KS0
    x1 = x1 + _KS1
    for i in range(5):
        rs = _ROT[:4] if i % 2 == 0 else _ROT[4:]
        for r in rs:
            x0 = x0 + x1
            x1 = (x1 << jnp.uint32(r)) | (x1 >> jnp.uint32(32 - r))
            x1 = x0 ^ x1
        x0 = x0 + ks[(i + 1) % 3]
        x1 = x1 + ks[(i + 2) % 3] + jnp.uint32(i + 1)
    return x0 ^ x1


def _noise(idx_u32):
    bits = _threefry_bits(idx_u32)
    float_bits = (bits >> jnp.uint32(9)) | jnp.uint32(0x3F800000)
    f = jax.lax.bitcast_convert_type(float_bits, jnp.float32) - jnp.float32(1.0)
    u = jnp.maximum(_LO, f * _SPAN + _LO)
    return _SQRT2 * jax.lax.erf_inv(u)


def _body(zfar_ref, znear_ref, zb_ref, pm_ref, mk_ref, out_ref, zm_ref, *,
          rt, sstride):
    k = zb_ref.shape[0]
    s = pl.program_id(1)

    @pl.when(s == 0)
    def _():
        zfar = zfar_ref[0]
        denom = zfar - znear_ref[0]
        z_inv = [(zfar - zb_ref[c]) / denom * mk_ref[c] for c in range(k)]
        zmax = z_inv[0]
        for c in range(1, k):
            zmax = jnp.maximum(zmax, z_inv[c])
        zmax = jnp.maximum(zmax, _EPS)
        for c in range(k):
            zm_ref[c] = _GAMMA * jnp.log(pm_ref[c]) + z_inv[c] - zmax
        zm_ref[k] = _EPS - zmax

    z_map = [zm_ref[c] for c in range(k + 1)]

    row = jax.lax.broadcasted_iota(jnp.int32, (rt, 128), 0)
    lane = jax.lax.broadcasted_iota(jnp.int32, (rt, 128), 1)
    p = (pl.program_id(0) * rt + row) * 128 + lane
    base = (p * (k + 1) + s * sstride).astype(jnp.uint32)

    best = None
    bidx = None
    for c in range(k + 1):
        n = _noise(base + jnp.uint32(c))
        v = z_map[c] + _GAMMA * n
        if c == 0:
            best = v
            bidx = jnp.zeros_like(row)
        else:
            upd = v > best
            best = jnp.where(upd, v, best)
            bidx = jnp.where(upd, c, bidx)

    last = jnp.int32(_NB_SAMPLES - 1)
    for c in range(k + 1):
        w = (bidx == c).astype(jnp.float32)

        @pl.when(s == 0)
        def _():
            out_ref[c] = w

        @pl.when(jnp.logical_and(s > 0, s < last))
        def _():
            out_ref[c] += w

        @pl.when(s == last)
        def _():
            out_ref[c] = (out_ref[c] + w) * jnp.float32(1.0 / _NB_SAMPLES)


def kernel(zbuf, zfar, znear, prob_map, mask):
    b, h, w, k = zbuf.shape
    p = b * h * w
    r = p // 128
    rt = 56 if r % 56 == 0 else 8
    zb = zbuf.reshape(p, k).T.reshape(k, r, 128)
    pm = prob_map.reshape(p, k).T.reshape(k, r, 128)
    mk = mask.reshape(p, k).T.reshape(k, r, 128)

    body = functools.partial(_body, rt=rt, sstride=p * (k + 1))
    out = pl.pallas_call(
        body,
        grid=(r // rt, _NB_SAMPLES),
        in_specs=[
            pl.BlockSpec(memory_space=pltpu.SMEM),
            pl.BlockSpec(memory_space=pltpu.SMEM),
            pl.BlockSpec((k, rt, 128), lambda i, s: (0, i, 0)),
            pl.BlockSpec((k, rt, 128), lambda i, s: (0, i, 0)),
            pl.BlockSpec((k, rt, 128), lambda i, s: (0, i, 0)),
        ],
        out_specs=pl.BlockSpec((k + 1, rt, 128), lambda i, s: (0, i, 0)),
        out_shape=jax.ShapeDtypeStruct((k + 1, r, 128), jnp.float32),
        scratch_shapes=[pltpu.VMEM((k + 1, rt, 128), jnp.float32)],
    )(zfar, znear, zb, pm, mk)
    return out.reshape(k + 1, p).T.reshape(b, h, w, k + 1)


# speedup vs baseline: 1.0960x; 1.0602x over previous
import functools

import jax
import jax.numpy as jnp
import numpy as np
from jax.experimental import pallas as pl
from jax.experimental.pallas import tpu as pltpu

_NB_SAMPLES = 16
_GAMMA = np.float32(0.04)
_EPS = np.float32(1e-10)
_LO = np.float32(np.nextafter(np.float32(-1.0), np.float32(0.0)))
_SPAN = np.float32(np.float32(1.0) - _LO)
_SQRT2 = np.float32(np.sqrt(2.0))

_KS0 = np.uint32(0)
_KS1 = np.uint32(1)
_KS2 = np.uint32(0x1BD11BDA) ^ _KS0 ^ _KS1
_ROT = (13, 15, 26, 6, 17, 29, 16, 24)


def _threefry_bits(x1):
    ks = (_KS0, _KS1, _KS2)
    x0 = jnp.zeros_like(x1) + _KS0
    x1 = x1 + _KS1
    for i in range(5):
        rs = _ROT[:4] if i % 2 == 0 else _ROT[4:]
        for r in rs:
            x0 = x0 + x1
            x1 = (x1 << jnp.uint32(r)) | (x1 >> jnp.uint32(32 - r))
            x1 = x0 ^ x1
        x0 = x0 + ks[(i + 1) % 3]
        x1 = x1 + ks[(i + 2) % 3] + jnp.uint32(i + 1)
    return x0 ^ x1


_ERFINV_LT = [
    2.81022636e-08, 3.43273939e-07, -3.5233877e-06,
    -4.39150654e-06, 0.00021858087, -0.00125372503,
    -0.00417768164, 0.246640727, 1.50140941,
]
_ERFINV_GT = [
    -0.000200214257, 0.000100950558, 0.00134934322,
    -0.00367342844, 0.00573950773, -0.0076224613,
    0.00943887047, 1.00167406, 2.83297682,
]


def _erf_inv(x):
    w = -jnp.log1p(x * -x)
    lt = w < 5.0
    w = jnp.where(lt, w - 2.5, jnp.sqrt(w) - 3.0)
    p = jnp.where(lt, np.float32(_ERFINV_LT[0]), np.float32(_ERFINV_GT[0]))
    for a, b in zip(_ERFINV_LT[1:], _ERFINV_GT[1:]):
        c = jnp.where(lt, np.float32(a), np.float32(b))
        p = c + p * w
    return p * x


def _noise(idx_u32):
    bits = _threefry_bits(idx_u32)
    float_bits = (bits >> jnp.uint32(9)) | jnp.uint32(0x3F800000)
    f = jax.lax.bitcast_convert_type(float_bits, jnp.float32) - jnp.float32(1.0)
    u = f * _SPAN + _LO
    return _SQRT2 * _erf_inv(u)


def _body(zfar_ref, znear_ref, zb_ref, pm_ref, mk_ref, out_ref, zm_ref, *,
          rt, sstride):
    k = zb_ref.shape[0]
    s = pl.program_id(1)

    @pl.when(s == 0)
    def _():
        zfar = zfar_ref[0]
        denom = zfar - znear_ref[0]
        z_inv = [(zfar - zb_ref[c]) / denom * mk_ref[c] for c in range(k)]
        zmax = z_inv[0]
        for c in range(1, k):
            zmax = jnp.maximum(zmax, z_inv[c])
        zmax = jnp.maximum(zmax, _EPS)
        for c in range(k):
            zm_ref[c] = _GAMMA * jnp.log(pm_ref[c]) + z_inv[c] - zmax
        zm_ref[k] = _EPS - zmax

    z_map = [zm_ref[c] for c in range(k + 1)]

    row = jax.lax.broadcasted_iota(jnp.int32, (rt, 128), 0)
    lane = jax.lax.broadcasted_iota(jnp.int32, (rt, 128), 1)
    p = (pl.program_id(0) * rt + row) * 128 + lane
    base = (p * (k + 1) + s * sstride).astype(jnp.uint32)

    best = None
    bidx = None
    for c in range(k + 1):
        n = _noise(base + jnp.uint32(c))
        v = z_map[c] + _GAMMA * n
        if c == 0:
            best = v
            bidx = jnp.zeros_like(row)
        else:
            upd = v > best
            best = jnp.where(upd, v, best)
            bidx = jnp.where(upd, c, bidx)

    last = jnp.int32(_NB_SAMPLES - 1)
    for c in range(k + 1):
        w = (bidx == c).astype(jnp.float32)

        @pl.when(s == 0)
        def _():
            out_ref[c] = w

        @pl.when(jnp.logical_and(s > 0, s < last))
        def _():
            out_ref[c] += w

        @pl.when(s == last)
        def _():
            out_ref[c] = (out_ref[c] + w) * jnp.float32(1.0 / _NB_SAMPLES)


def kernel(zbuf, zfar, znear, prob_map, mask):
    b, h, w, k = zbuf.shape
    p = b * h * w
    r = p // 128
    rt = 112 if r % 112 == 0 else 8
    zb = zbuf.reshape(p, k).T.reshape(k, r, 128)
    pm = prob_map.reshape(p, k).T.reshape(k, r, 128)
    mk = mask.reshape(p, k).T.reshape(k, r, 128)

    body = functools.partial(_body, rt=rt, sstride=p * (k + 1))
    out = pl.pallas_call(
        body,
        grid=(r // rt, _NB_SAMPLES),
        in_specs=[
            pl.BlockSpec(memory_space=pltpu.SMEM),
            pl.BlockSpec(memory_space=pltpu.SMEM),
            pl.BlockSpec((k, rt, 128), lambda i, s: (0, i, 0)),
            pl.BlockSpec((k, rt, 128), lambda i, s: (0, i, 0)),
            pl.BlockSpec((k, rt, 128), lambda i, s: (0, i, 0)),
        ],
        out_specs=pl.BlockSpec((k + 1, rt, 128), lambda i, s: (0, i, 0)),
        out_shape=jax.ShapeDtypeStruct((k + 1, r, 128), jnp.float32),
        scratch_shapes=[pltpu.VMEM((k + 1, rt, 128), jnp.float32)],
    )(zfar, znear, zb, pm, mk)
    return out.reshape(k + 1, p).T.reshape(b, h, w, k + 1)
